# baseline (speedup 1.0000x reference)
import functools

import jax
import jax.numpy as jnp
from jax import lax
from jax.experimental import pallas as pl
from jax.experimental.pallas import tpu as pltpu
from jax.experimental.pallas import tpu_sc as plsc

HID = 128
EA = 4
NT = 5
NG = 64
N_NODES = 10000
N_EDGES = 320000

B_E = 256
NBLK = N_EDGES // B_E
NW = 32
CHUNK = N_EDGES // NW
SUB = 80
KSUB = 5
OUTER = SUB * KSUB
NTILES = CHUNK // OUTER
NTI = NW * NTILES
SCT = N_EDGES // OUTER
HROW = 5200
ACC_R = 5248
STRIPE = ACC_R // 16
R_N = 400
NB_N = N_NODES // R_N
PB = HROW // R_N

_MESH = dict(core_axis_name="c", subcore_axis_name="s")


def _sc_gather(h, idx2, ntg):
    @functools.partial(
        pl.kernel, mesh=plsc.VectorSubcoreMesh(**_MESH),
        out_type=jax.ShapeDtypeStruct((N_EDGES, HID), jnp.float32),
        scratch_types=[
            pltpu.VMEM((KSUB, SUB), jnp.int32),
            pltpu.VMEM((1, 8), jnp.int32),
            pltpu.VMEM((OUTER, HID), jnp.float32),
            pltpu.SemaphoreType.DMA,
        ],
    )
    def k(h_hbm, idx_hbm, ntg_hbm, out_hbm, idx_v, ntv, rows_v, sem):
        wid = lax.axis_index("s") * 2 + lax.axis_index("c")
        pltpu.sync_copy(ntg_hbm.at[wid], ntv)
        nloc = ntv[0][0]

        def body(t, carry):
            g = wid + NW * t
            pltpu.sync_copy(idx_hbm.at[g], idx_v)
            cps = [pltpu.async_copy(h_hbm.at[idx_v.at[j]],
                                    rows_v.at[pl.ds(j * SUB, SUB)], sem)
                   for j in range(KSUB)]
            for cp in cps:
                cp.wait()
            pltpu.sync_copy(rows_v, out_hbm.at[pl.ds(g * OUTER, OUTER)])
            return carry

        lax.fori_loop(0, nloc, body, 0)

    return k(h, idx2, ntg)


def _sc_scatter(m, mpos2, ldst2, nts, zeros_msg):
    @functools.partial(
        pl.kernel, mesh=plsc.VectorSubcoreMesh(**_MESH),
        out_type=jax.ShapeDtypeStruct((2, ACC_R, HID), jnp.float32),
        scratch_types=[
            pltpu.VMEM((KSUB, SUB), jnp.int32),
            pltpu.VMEM((KSUB, SUB), jnp.int32),
            pltpu.VMEM((1, 8), jnp.int32),
            pltpu.VMEM((OUTER, HID), jnp.float32),
            pltpu.VMEM_SHARED((ACC_R, HID), jnp.float32),
            pltpu.SemaphoreType.DMA,
        ],
    )
    def k(m_hbm, mpos_hbm, ldst_hbm, nt_hbm, z_hbm, out_hbm,
          idx_v, dst_v, ntv, rows_v, acc, sem):
        cid = lax.axis_index("c")
        sid = lax.axis_index("s")

        pltpu.sync_copy(z_hbm.at[pl.ds(sid * STRIPE, STRIPE)],
                        acc.at[pl.ds(sid * STRIPE, STRIPE)])
        pltpu.sync_copy(nt_hbm.at[cid], ntv)
        plsc.subcore_barrier()
        nloc = (ntv[0][0] - sid + 15) // 16

        def body(t, carry):
            tile = sid + 16 * t
            pltpu.sync_copy(mpos_hbm.at[cid, tile], idx_v)
            pltpu.sync_copy(ldst_hbm.at[cid, tile], dst_v)
            cps = [pltpu.async_copy(m_hbm.at[idx_v.at[j]],
                                    rows_v.at[pl.ds(j * SUB, SUB)], sem)
                   for j in range(KSUB)]
            for cp in cps:
                cp.wait()
            cps = [pltpu.async_copy(rows_v.at[pl.ds(j * SUB, SUB)],
                                    acc.at[dst_v.at[j]], sem, add=True)
                   for j in range(KSUB)]
            for cp in cps:
                cp.wait()
            return carry

        lax.fori_loop(0, nloc, body, 0)
        plsc.subcore_barrier()
        pltpu.sync_copy(acc.at[pl.ds(sid * STRIPE, STRIPE)],
                        out_hbm.at[cid, pl.ds(sid * STRIPE, STRIPE)])

    return k(m, mpos2, ldst2, nts, zeros_msg)


def _mlp_blocks(hsrc, rec_s, w1h, w1a, b1, w2, b2, bt):
    def kern(bt_ref, hs_ref, rec_ref, w1h_r, w1a_r, b1_r, w2_r, b2_r, out):
        k = pl.program_id(0)
        t = bt_ref[k]
        ea = rec_ref[...][:, :EA]

        def mlp_t(tt):
            h1 = hs_ref[...] @ w1h_r[tt] + ea @ w1a_r[tt] + b1_r[tt]
            return jnp.maximum(h1, 0.0) @ w2_r[tt] + b2_r[tt]

        @pl.when(t >= 0)
        def _():
            out[...] = mlp_t(t)

        @pl.when(t == -1)
        def _():
            typef = rec_ref[...][:, 7:8]
            acc = jnp.zeros((B_E, HID), jnp.float32)
            for tt in range(NT):
                acc = acc + jnp.where(typef == float(tt), mlp_t(tt), 0.0)
            out[...] = acc

    return pl.pallas_call(
        kern,
        grid_spec=pltpu.PrefetchScalarGridSpec(
            num_scalar_prefetch=1,
            grid=(NBLK,),
            in_specs=[
                pl.BlockSpec((B_E, HID),
                             lambda i, bt: (jnp.where(bt[i] == -2, 0, i), 0)),
                pl.BlockSpec((B_E, 8),
                             lambda i, bt: (jnp.where(bt[i] == -2, 0, i), 0)),
                pl.BlockSpec((NT, HID, HID), lambda i, bt: (0, 0, 0)),
                pl.BlockSpec((NT, EA, HID), lambda i, bt: (0, 0, 0)),
                pl.BlockSpec((NT, 1, HID), lambda i, bt: (0, 0, 0)),
                pl.BlockSpec((NT, HID, HID), lambda i, bt: (0, 0, 0)),
                pl.BlockSpec((NT, 1, HID), lambda i, bt: (0, 0, 0)),
            ],
            out_specs=pl.BlockSpec(
                (B_E, HID), lambda i, bt: (jnp.where(bt[i] == -2, NBLK, i), 0)),
        ),
        out_shape=jax.ShapeDtypeStruct(((NBLK + 1) * B_E, HID), jnp.float32),
        compiler_params=pltpu.CompilerParams(
            dimension_semantics=("arbitrary",)),
    )(bt, hsrc, rec_s, w1h, w1a, b1, w2, b2)


def _gru_blocks(h, msgs, wihT, whhT, bih, bhh):
    def kern(h_r, m_r, wih_r, whh_r, bi_r, bh_r, out):
        msg = m_r[0]
        gi = msg @ wih_r[...] + bi_r[...]
        gh = h_r[...] @ whh_r[...] + bh_r[...]
        r = jax.nn.sigmoid(gi[:, :HID] + gh[:, :HID])
        z = jax.nn.sigmoid(gi[:, HID:2 * HID] + gh[:, HID:2 * HID])
        n = jnp.tanh(gi[:, 2 * HID:] + r * gh[:, 2 * HID:])
        out[...] = (1.0 - z) * n + z * h_r[...]

    return pl.pallas_call(
        kern,
        grid=(NB_N,),
        in_specs=[
            pl.BlockSpec((R_N, HID), lambda i: (i, 0)),
            pl.BlockSpec((1, R_N, HID), lambda i: (i // PB, i % PB, 0)),
            pl.BlockSpec((HID, 3 * HID), lambda i: (0, 0)),
            pl.BlockSpec((HID, 3 * HID), lambda i: (0, 0)),
            pl.BlockSpec((1, 3 * HID), lambda i: (0, 0)),
            pl.BlockSpec((1, 3 * HID), lambda i: (0, 0)),
        ],
        out_specs=pl.BlockSpec((R_N, HID), lambda i: (i, 0)),
        out_shape=jax.ShapeDtypeStruct((N_NODES, HID), jnp.float32),
        compiler_params=pltpu.CompilerParams(
            dimension_semantics=("arbitrary",)),
    )(h, msgs, wihT, whhT, bih, bhh)


def _readout(h, seg3d, w1T, b1, w2T, b2):
    def kern(h_r, seg_r, w1_r, b1_r, w2_r, b2_r, out, acc):
        i = pl.program_id(0)

        @pl.when(i == 0)
        def _():
            acc[...] = jnp.zeros_like(acc)

        seg = seg_r[0]
        row = lax.broadcasted_iota(jnp.int32, (NG, R_N), 0)
        onehot = (row == seg).astype(jnp.float32)
        acc[...] += onehot @ h_r[...]

        @pl.when(i == NB_N - 1)
        def _():
            g = jnp.maximum(acc[...] @ w1_r[...] + b1_r[...], 0.0)
            out[...] = g @ w2_r[...] + b2_r[0, 0]

    return pl.pallas_call(
        kern,
        grid=(NB_N,),
        in_specs=[
            pl.BlockSpec((R_N, HID), lambda i: (i, 0)),
            pl.BlockSpec((1, 1, R_N), lambda i: (i, 0, 0)),
            pl.BlockSpec((HID, HID), lambda i: (0, 0)),
            pl.BlockSpec((1, HID), lambda i: (0, 0)),
            pl.BlockSpec((HID, 1), lambda i: (0, 0)),
            pl.BlockSpec((1, 1), lambda i: (0, 0)),
        ],
        out_specs=pl.BlockSpec((NG, 1), lambda i: (0, 0)),
        out_shape=jax.ShapeDtypeStruct((NG, 1), jnp.float32),
        scratch_shapes=[pltpu.VMEM((NG, HID), jnp.float32)],
        compiler_params=pltpu.CompilerParams(
            dimension_semantics=("arbitrary",)),
    )(h, seg3d, w1T, b1, w2T, b2)


def _pack_mlps(mlps, ntypes):
    w1h = jnp.zeros((NT, HID, HID), jnp.float32)
    w1a = jnp.zeros((NT, EA, HID), jnp.float32)
    b1 = jnp.zeros((NT, 1, HID), jnp.float32)
    w2 = jnp.zeros((NT, HID, HID), jnp.float32)
    b2 = jnp.zeros((NT, 1, HID), jnp.float32)
    for t in range(ntypes):
        mp = mlps[t]
        w1h = w1h.at[t].set(mp["W1"][:, :HID].T)
        w1a = w1a.at[t].set(mp["W1"][:, HID:].T)
        b1 = b1.at[t, 0].set(mp["b1"])
        w2 = w2.at[t].set(mp["W2"].T)
        b2 = b2.at[t, 0].set(mp["b2"])
    return w1h, w1a, b1, w2, b2


def _pack_gru(g):
    return (g["Wih"].T, g["Whh"].T,
            g["bih"].reshape(1, 3 * HID), g["bhh"].reshape(1, 3 * HID))


def kernel(x, edge_index, edge_type, edge_attr, batch, params):
    src, dst = edge_index[0], edge_index[1]
    et = edge_type.astype(jnp.int32)

    tids = jnp.arange(NT, dtype=jnp.int32)
    type_eq = et[None, :] == tids[:, None]
    counts = type_eq.sum(axis=1).astype(jnp.int32)
    rank = (jnp.cumsum(type_eq, axis=1) - 1).astype(jnp.int32)
    rank = jnp.where(type_eq, rank, 0).sum(axis=0)
    cum = jnp.cumsum(counts)
    base = cum - counts
    pos = base[et] + rank

    as_f = lambda a: lax.bitcast_convert_type(a.astype(jnp.int32), jnp.float32)
    rec = jnp.concatenate(
        [edge_attr, as_f(src)[:, None], as_f(dst)[:, None],
         as_f(dst)[:, None], et.astype(jnp.float32)[:, None]], axis=1)
    inv = jnp.zeros((N_EDGES,), jnp.int32).at[pos].set(
        jnp.arange(N_EDGES, dtype=jnp.int32))
    rec_s = jnp.take(rec, inv, axis=0)

    as_i = lambda a: lax.bitcast_convert_type(a, jnp.int32)
    src2 = as_i(rec_s[:, 4]).reshape(NTI, KSUB, SUB)
    dst_s = as_i(rec_s[:, 5])
    type_s = rec_s[:, 7]

    c0 = counts[0]
    wids = jnp.arange(NW, dtype=jnp.int32)

    def _gather_counts(nedges):
        ntiles = (nedges + OUTER - 1) // OUTER
        cnt = (ntiles - wids + NW - 1) // NW
        return jnp.broadcast_to(cnt[:, None, None], (NW, 1, 8)).astype(jnp.int32)

    ntg_sp = _gather_counts(jnp.int32(N_EDGES))
    ntg_b = _gather_counts(c0)

    ar = jnp.arange(N_EDGES, dtype=jnp.int32)
    trash_p = HROW + (ar // OUTER) % 16

    def _part(valid):
        mls, dls, nts = [], [], []
        for c in range(2):
            lo = c * HROW
            own = valid & (dst_s >= lo) & (dst_s < lo + HROW)
            n_c = own.sum().astype(jnp.int32)
            pos_own = jnp.cumsum(own).astype(jnp.int32) - 1
            pos_rest = n_c + jnp.cumsum(~own).astype(jnp.int32) - 1
            dest = jnp.where(own, pos_own, pos_rest)
            mpos = jnp.zeros((N_EDGES,), jnp.int32).at[dest].set(ar)
            ld = jnp.zeros((N_EDGES,), jnp.int32).at[dest].set(dst_s - lo)
            ld = jnp.where(ar < n_c, ld, trash_p)
            mls.append(mpos.reshape(SCT, KSUB, SUB))
            dls.append(ld.reshape(SCT, KSUB, SUB))
            nts.append(jnp.broadcast_to((n_c + OUTER - 1) // OUTER, (1, 8)))
        return (jnp.stack(mls), jnp.stack(dls),
                jnp.stack(nts).astype(jnp.int32))

    sc_sp = _part(jnp.ones((N_EDGES,), jnp.bool_))
    sc_b = _part(type_s == 0.0)

    blo = jnp.arange(NBLK, dtype=jnp.int32) * B_E
    t_lo = jnp.searchsorted(cum, blo, side="right").astype(jnp.int32)
    t_hi = jnp.searchsorted(cum, blo + (B_E - 1), side="right").astype(jnp.int32)
    bt_sp = jnp.where(t_lo == t_hi, t_lo, -1).astype(jnp.int32)
    bt_bond = jnp.where(blo + B_E <= c0, 0,
                        jnp.where(blo < c0, -1, -2)).astype(jnp.int32)

    zeros_msg = jnp.zeros((ACC_R, HID), jnp.float32)
    seg3d = batch.astype(jnp.int32).reshape(NB_N, 1, R_N)

    def layer(h, mlp_pack, gru_pack, bt, ntg, sc):
        hsrc = _sc_gather(h, src2, ntg)
        m = _mlp_blocks(hsrc, rec_s, *mlp_pack, bt)
        msgs = _sc_scatter(m, *sc, zeros_msg)
        return _gru_blocks(h, msgs, *gru_pack)

    bond_packs = [(_pack_mlps(lp["mlps"], 1), _pack_gru(lp["gru"]))
                  for lp in params["bond"]]
    spatial_packs = [(_pack_mlps(lp["mlps"], NT), _pack_gru(lp["gru"]))
                     for lp in params["spatial"]]

    def bond_branch(hh):
        for mp, gp in bond_packs:
            hh = layer(hh, mp, gp, bt_bond, ntg_b, sc_b)
        return hh

    h = lax.cond(c0 > 0, bond_branch, lambda hh: hh, x)
    for mp, gp in spatial_packs:
        h = layer(h, mp, gp, bt_sp, ntg_sp, sc_sp)

    r = params["readout"]
    out = _readout(h, seg3d, r["W1"].T, r["b1"].reshape(1, HID),
                   r["W2"].T, r["b2"].reshape(1, 1))
    return out.reshape(-1)


# speedup vs baseline: 2.4647x; 2.4647x over previous
import functools

import jax
import jax.numpy as jnp
from jax import lax
from jax.experimental import pallas as pl
from jax.experimental.pallas import tpu as pltpu
from jax.experimental.pallas import tpu_sc as plsc

HID = 128
EA = 4
NT = 5
NG = 64
N_NODES = 10000
N_EDGES = 320000

B_E = 256
NBLK = N_EDGES // B_E
NW = 32
CHUNK = N_EDGES // NW
SUB = 80
KSUB = 5
OUTER = SUB * KSUB
NTILES = CHUNK // OUTER
NTI = NW * NTILES
HROW = 5200
ACC_R = 5248
STRIPE = ACC_R // 16
R_N = 400
NB_N = N_NODES // R_N
PB = HROW // R_N

_MESH = dict(core_axis_name="c", subcore_axis_name="s")


def _sc_gather(h, idx2, ntg):
    @functools.partial(
        pl.kernel, mesh=plsc.VectorSubcoreMesh(**_MESH),
        out_type=jax.ShapeDtypeStruct((N_EDGES, HID), jnp.float32),
        scratch_types=[
            pltpu.VMEM((KSUB, SUB), jnp.int32),
            pltpu.VMEM((1, 8), jnp.int32),
            pltpu.VMEM((OUTER, HID), jnp.float32),
            pltpu.SemaphoreType.DMA,
        ],
    )
    def k(h_hbm, idx_hbm, ntg_hbm, out_hbm, idx_v, ntv, rows_v, sem):
        wid = lax.axis_index("s") * 2 + lax.axis_index("c")
        pltpu.sync_copy(ntg_hbm.at[wid], ntv)
        nloc = ntv[0][0]

        def body(t, carry):
            g = wid + NW * t
            pltpu.sync_copy(idx_hbm.at[g], idx_v)
            cps = [pltpu.async_copy(h_hbm.at[idx_v.at[j]],
                                    rows_v.at[pl.ds(j * SUB, SUB)], sem)
                   for j in range(KSUB)]
            for cp in cps:
                cp.wait()
            pltpu.sync_copy(rows_v, out_hbm.at[pl.ds(g * OUTER, OUTER)])
            return carry

        lax.fori_loop(0, nloc, body, 0)

    return k(h, idx2, ntg)


def _sc_scatter(m, dst2, ntg, zeros_msg):
    @functools.partial(
        pl.kernel, mesh=plsc.VectorSubcoreMesh(**_MESH),
        out_type=jax.ShapeDtypeStruct((2, 2, ACC_R, HID), jnp.float32),
        scratch_types=[
            pltpu.VMEM((KSUB, SUB), jnp.int32),
            pltpu.VMEM((1, 8), jnp.int32),
            pltpu.VMEM((OUTER, HID), jnp.float32),
            pltpu.VMEM_SHARED((ACC_R, HID), jnp.float32),
            pltpu.SemaphoreType.DMA,
        ],
    )
    def k(m_hbm, dst_hbm, ntg_hbm, z_hbm, out_hbm,
          dst_v, ntv, rows_v, acc, sem):
        cid = lax.axis_index("c")
        sid = lax.axis_index("s")
        wid = sid * 2 + cid
        pltpu.sync_copy(ntg_hbm.at[wid], ntv)
        nloc = ntv[0][0]

        for p in range(2):
            pltpu.sync_copy(z_hbm.at[pl.ds(sid * STRIPE, STRIPE)],
                            acc.at[pl.ds(sid * STRIPE, STRIPE)])
            plsc.subcore_barrier()

            def body(t, carry):
                g = wid + NW * t
                pltpu.sync_copy(dst_hbm.at[p, g], dst_v)
                pltpu.sync_copy(m_hbm.at[pl.ds(g * OUTER, OUTER)], rows_v)
                cps = [pltpu.async_copy(rows_v.at[pl.ds(j * SUB, SUB)],
                                        acc.at[dst_v.at[j]], sem, add=True)
                       for j in range(KSUB)]
                for cp in cps:
                    cp.wait()
                return carry

            lax.fori_loop(0, nloc, body, 0)
            plsc.subcore_barrier()
            pltpu.sync_copy(acc.at[pl.ds(sid * STRIPE, STRIPE)],
                            out_hbm.at[p, cid, pl.ds(sid * STRIPE, STRIPE)])

    return k(m, dst2, ntg, zeros_msg)


def _mlp_blocks(hsrc, rec_s, w1h, w1a, b1, w2, b2, bt):
    def kern(bt_ref, hs_ref, rec_ref, w1h_r, w1a_r, b1_r, w2_r, b2_r, out):
        k = pl.program_id(0)
        t = bt_ref[k]
        ea = rec_ref[...][:, :EA]

        def mlp_t(tt):
            h1 = hs_ref[...] @ w1h_r[tt] + ea @ w1a_r[tt] + b1_r[tt]
            return jnp.maximum(h1, 0.0) @ w2_r[tt] + b2_r[tt]

        @pl.when(t >= 0)
        def _():
            out[...] = mlp_t(t)

        @pl.when(t == -1)
        def _():
            typef = rec_ref[...][:, 7:8]
            acc = jnp.zeros((B_E, HID), jnp.float32)
            for tt in range(NT):
                acc = acc + jnp.where(typef == float(tt), mlp_t(tt), 0.0)
            out[...] = acc

    return pl.pallas_call(
        kern,
        grid_spec=pltpu.PrefetchScalarGridSpec(
            num_scalar_prefetch=1,
            grid=(NBLK,),
            in_specs=[
                pl.BlockSpec((B_E, HID),
                             lambda i, bt: (jnp.where(bt[i] == -2, 0, i), 0)),
                pl.BlockSpec((B_E, 8),
                             lambda i, bt: (jnp.where(bt[i] == -2, 0, i), 0)),
                pl.BlockSpec((NT, HID, HID), lambda i, bt: (0, 0, 0)),
                pl.BlockSpec((NT, EA, HID), lambda i, bt: (0, 0, 0)),
                pl.BlockSpec((NT, 1, HID), lambda i, bt: (0, 0, 0)),
                pl.BlockSpec((NT, HID, HID), lambda i, bt: (0, 0, 0)),
                pl.BlockSpec((NT, 1, HID), lambda i, bt: (0, 0, 0)),
            ],
            out_specs=pl.BlockSpec(
                (B_E, HID), lambda i, bt: (jnp.where(bt[i] == -2, NBLK, i), 0)),
        ),
        out_shape=jax.ShapeDtypeStruct(((NBLK + 1) * B_E, HID), jnp.float32),
        compiler_params=pltpu.CompilerParams(
            dimension_semantics=("arbitrary",)),
    )(bt, hsrc, rec_s, w1h, w1a, b1, w2, b2)


def _gru_blocks(h, msgs, wihT, whhT, bih, bhh):
    def kern(h_r, m0_r, m1_r, wih_r, whh_r, bi_r, bh_r, out):
        msg = m0_r[0, 0] + m1_r[0, 0]
        gi = msg @ wih_r[...] + bi_r[...]
        gh = h_r[...] @ whh_r[...] + bh_r[...]
        r = jax.nn.sigmoid(gi[:, :HID] + gh[:, :HID])
        z = jax.nn.sigmoid(gi[:, HID:2 * HID] + gh[:, HID:2 * HID])
        n = jnp.tanh(gi[:, 2 * HID:] + r * gh[:, 2 * HID:])
        out[...] = (1.0 - z) * n + z * h_r[...]

    return pl.pallas_call(
        kern,
        grid=(NB_N,),
        in_specs=[
            pl.BlockSpec((R_N, HID), lambda i: (i, 0)),
            pl.BlockSpec((1, 1, R_N, HID), lambda i: (i // PB, 0, i % PB, 0)),
            pl.BlockSpec((1, 1, R_N, HID), lambda i: (i // PB, 1, i % PB, 0)),
            pl.BlockSpec((HID, 3 * HID), lambda i: (0, 0)),
            pl.BlockSpec((HID, 3 * HID), lambda i: (0, 0)),
            pl.BlockSpec((1, 3 * HID), lambda i: (0, 0)),
            pl.BlockSpec((1, 3 * HID), lambda i: (0, 0)),
        ],
        out_specs=pl.BlockSpec((R_N, HID), lambda i: (i, 0)),
        out_shape=jax.ShapeDtypeStruct((N_NODES, HID), jnp.float32),
        compiler_params=pltpu.CompilerParams(
            dimension_semantics=("arbitrary",)),
    )(h, msgs, msgs, wihT, whhT, bih, bhh)


def _readout(h, seg3d, w1T, b1, w2T, b2):
    def kern(h_r, seg_r, w1_r, b1_r, w2_r, b2_r, out, acc):
        i = pl.program_id(0)

        @pl.when(i == 0)
        def _():
            acc[...] = jnp.zeros_like(acc)

        seg = seg_r[0]
        row = lax.broadcasted_iota(jnp.int32, (NG, R_N), 0)
        onehot = (row == seg).astype(jnp.float32)
        acc[...] += onehot @ h_r[...]

        @pl.when(i == NB_N - 1)
        def _():
            g = jnp.maximum(acc[...] @ w1_r[...] + b1_r[...], 0.0)
            out[...] = g @ w2_r[...] + b2_r[0, 0]

    return pl.pallas_call(
        kern,
        grid=(NB_N,),
        in_specs=[
            pl.BlockSpec((R_N, HID), lambda i: (i, 0)),
            pl.BlockSpec((1, 1, R_N), lambda i: (i, 0, 0)),
            pl.BlockSpec((HID, HID), lambda i: (0, 0)),
            pl.BlockSpec((1, HID), lambda i: (0, 0)),
            pl.BlockSpec((HID, 1), lambda i: (0, 0)),
            pl.BlockSpec((1, 1), lambda i: (0, 0)),
        ],
        out_specs=pl.BlockSpec((NG, 1), lambda i: (0, 0)),
        out_shape=jax.ShapeDtypeStruct((NG, 1), jnp.float32),
        scratch_shapes=[pltpu.VMEM((NG, HID), jnp.float32)],
        compiler_params=pltpu.CompilerParams(
            dimension_semantics=("arbitrary",)),
    )(h, seg3d, w1T, b1, w2T, b2)


def _pack_mlps(mlps, ntypes):
    w1h = jnp.zeros((NT, HID, HID), jnp.float32)
    w1a = jnp.zeros((NT, EA, HID), jnp.float32)
    b1 = jnp.zeros((NT, 1, HID), jnp.float32)
    w2 = jnp.zeros((NT, HID, HID), jnp.float32)
    b2 = jnp.zeros((NT, 1, HID), jnp.float32)
    for t in range(ntypes):
        mp = mlps[t]
        w1h = w1h.at[t].set(mp["W1"][:, :HID].T)
        w1a = w1a.at[t].set(mp["W1"][:, HID:].T)
        b1 = b1.at[t, 0].set(mp["b1"])
        w2 = w2.at[t].set(mp["W2"].T)
        b2 = b2.at[t, 0].set(mp["b2"])
    return w1h, w1a, b1, w2, b2


def _pack_gru(g):
    return (g["Wih"].T, g["Whh"].T,
            g["bih"].reshape(1, 3 * HID), g["bhh"].reshape(1, 3 * HID))


def kernel(x, edge_index, edge_type, edge_attr, batch, params):
    src, dst = edge_index[0], edge_index[1]
    et = edge_type.astype(jnp.int32)

    tids = jnp.arange(NT, dtype=jnp.int32)
    type_eq = et[None, :] == tids[:, None]
    counts = type_eq.sum(axis=1).astype(jnp.int32)
    rank = (jnp.cumsum(type_eq, axis=1) - 1).astype(jnp.int32)
    rank = jnp.where(type_eq, rank, 0).sum(axis=0)
    cum = jnp.cumsum(counts)
    base = cum - counts
    pos = base[et] + rank

    as_f = lambda a: lax.bitcast_convert_type(a.astype(jnp.int32), jnp.float32)
    rec = jnp.concatenate(
        [edge_attr, as_f(src)[:, None], as_f(dst)[:, None],
         as_f(dst)[:, None], et.astype(jnp.float32)[:, None]], axis=1)
    inv = jnp.zeros((N_EDGES,), jnp.int32).at[pos].set(
        jnp.arange(N_EDGES, dtype=jnp.int32))
    rec_s = jnp.take(rec, inv, axis=0)

    as_i = lambda a: lax.bitcast_convert_type(a, jnp.int32)
    src2 = as_i(rec_s[:, 4]).reshape(NTI, KSUB, SUB)
    dst_s = as_i(rec_s[:, 5])
    type_s = rec_s[:, 7]

    c0 = counts[0]
    wids = jnp.arange(NW, dtype=jnp.int32)

    def _gather_counts(nedges):
        ntiles = (nedges + OUTER - 1) // OUTER
        cnt = (ntiles - wids + NW - 1) // NW
        return jnp.broadcast_to(cnt[:, None, None], (NW, 1, 8)).astype(jnp.int32)

    ntg_sp = _gather_counts(jnp.int32(N_EDGES))
    ntg_b = _gather_counts(c0)

    ar = jnp.arange(N_EDGES, dtype=jnp.int32)
    trash_l = HROW + (ar // OUTER) % 32

    def _local_dst(valid):
        outs = []
        for p in range(2):
            lo = p * HROW
            inr = valid & (dst_s >= lo) & (dst_s < lo + HROW)
            outs.append(jnp.where(inr, dst_s - lo, trash_l))
        return jnp.stack(outs).reshape(2, NTI, KSUB, SUB)

    dst2 = _local_dst(jnp.ones((N_EDGES,), jnp.bool_))
    dstb2 = _local_dst(type_s == 0.0)

    blo = jnp.arange(NBLK, dtype=jnp.int32) * B_E
    t_lo = jnp.searchsorted(cum, blo, side="right").astype(jnp.int32)
    t_hi = jnp.searchsorted(cum, blo + (B_E - 1), side="right").astype(jnp.int32)
    bt_sp = jnp.where(t_lo == t_hi, t_lo, -1).astype(jnp.int32)
    bt_bond = jnp.where(blo + B_E <= c0, 0,
                        jnp.where(blo < c0, -1, -2)).astype(jnp.int32)

    zeros_msg = jnp.zeros((ACC_R, HID), jnp.float32)
    seg3d = batch.astype(jnp.int32).reshape(NB_N, 1, R_N)

    def layer(h, mlp_pack, gru_pack, bt, ntg, d2):
        hsrc = _sc_gather(h, src2, ntg)
        m = _mlp_blocks(hsrc, rec_s, *mlp_pack, bt)
        msgs = _sc_scatter(m, d2, ntg, zeros_msg)
        return _gru_blocks(h, msgs, *gru_pack)

    bond_packs = [(_pack_mlps(lp["mlps"], 1), _pack_gru(lp["gru"]))
                  for lp in params["bond"]]
    spatial_packs = [(_pack_mlps(lp["mlps"], NT), _pack_gru(lp["gru"]))
                     for lp in params["spatial"]]

    def bond_branch(hh):
        for mp, gp in bond_packs:
            hh = layer(hh, mp, gp, bt_bond, ntg_b, dstb2)
        return hh

    h = lax.cond(c0 > 0, bond_branch, lambda hh: hh, x)
    for mp, gp in spatial_packs:
        h = layer(h, mp, gp, bt_sp, ntg_sp, dst2)

    r = params["readout"]
    out = _readout(h, seg3d, r["W1"].T, r["b1"].reshape(1, HID),
                   r["W2"].T, r["b2"].reshape(1, 1))
    return out.reshape(-1)
